# Initial kernel scaffold; baseline (speedup 1.0000x reference)
import jax
import jax.numpy as jnp
from jax.experimental import pallas as pl


def kernel(x, token_table, pos_table):
    raise NotImplementedError("write your pallas kernel here")


# speedup vs baseline: 1.1830x; 1.1830x over previous
import functools

import jax
import jax.numpy as jnp
from jax import lax
from jax.experimental import pallas as pl
from jax.experimental.pallas import tpu as pltpu
from jax.experimental.pallas import tpu_sc as plsc

VOCAB = 1000000
MAXLEN = 200
EMBED = 32
BATCH = 4096
SEQ = 200

NC = 2
NS = 16
NW = NC * NS
TOTAL = BATCH * SEQ
PER_W = TOTAL // NW
GATHER_ROWS = 128
NGATHER = 8
CHUNK = GATHER_ROWS * NGATHER
NCHUNK = PER_W // CHUNK

_mesh = plsc.VectorSubcoreMesh(core_axis_name="c", subcore_axis_name="s")


@functools.partial(
    pl.kernel,
    out_type=jax.ShapeDtypeStruct((TOTAL, EMBED), jnp.float32),
    mesh=_mesh,
    scratch_types=[
        pltpu.VMEM((NGATHER, GATHER_ROWS), jnp.int32),
        pltpu.VMEM((CHUNK, EMBED), jnp.float32),
        pltpu.VMEM((MAXLEN, EMBED), jnp.float32),
        pltpu.SemaphoreType.DMA,
    ],
    compiler_params=pltpu.CompilerParams(use_tc_tiling_on_sc=False),
)
def _embed_kernel(idx_hbm, table_hbm, pos_hbm, out_hbm, idx_v, rows_v, pos_v, sem):
    wid = lax.axis_index("s") * NC + lax.axis_index("c")
    base = wid * PER_W

    pltpu.sync_copy(pos_hbm, pos_v)

    def do_chunk(ci, carry):
        off = base + ci * CHUNK

        idx_row = pl.multiple_of(off // GATHER_ROWS, 8)
        pltpu.sync_copy(idx_hbm.at[pl.ds(idx_row, NGATHER)], idx_v)

        copies = []
        for g in range(NGATHER):
            copies.append(
                pltpu.async_copy(
                    table_hbm.at[idx_v.at[g]],
                    rows_v.at[pl.ds(g * GATHER_ROWS, GATHER_ROWS)],
                    sem,
                )
            )
        for c in copies:
            c.wait()

        def add_row(i, carry2):
            p = lax.rem(off + i, SEQ)
            rows_v[i, pl.ds(0, 16)] = rows_v[i, pl.ds(0, 16)] + pos_v[p, pl.ds(0, 16)]
            rows_v[i, pl.ds(16, 16)] = rows_v[i, pl.ds(16, 16)] + pos_v[p, pl.ds(16, 16)]
            return carry2

        lax.fori_loop(0, CHUNK, add_row, None, unroll=8)

        pltpu.sync_copy(rows_v, out_hbm.at[pl.ds(off, CHUNK)])
        return carry

    lax.fori_loop(0, NCHUNK, do_chunk, None)


def kernel(x, token_table, pos_table):
    idx = x.reshape(TOTAL // GATHER_ROWS, GATHER_ROWS).astype(jnp.int32)
    out = _embed_kernel(idx, token_table, pos_table)
    return out.reshape(BATCH, SEQ, EMBED)


# speedup vs baseline: 1.2336x; 1.0428x over previous
import functools

import jax
import jax.numpy as jnp
from jax import lax
from jax.experimental import pallas as pl
from jax.experimental.pallas import tpu as pltpu
from jax.experimental.pallas import tpu_sc as plsc

VOCAB = 1000000
MAXLEN = 200
EMBED = 32
BATCH = 4096
SEQ = 200

NC = 2
NS = 16
NW = NC * NS
TOTAL = BATCH * SEQ
PER_W = TOTAL // NW
GATHER_ROWS = 128
NGATHER = 10
CHUNK = GATHER_ROWS * NGATHER
NCHUNK = PER_W // CHUNK

_mesh = plsc.VectorSubcoreMesh(core_axis_name="c", subcore_axis_name="s")


@functools.partial(
    pl.kernel,
    out_type=jax.ShapeDtypeStruct((TOTAL, EMBED), jnp.float32),
    mesh=_mesh,
    scratch_types=[
        pltpu.VMEM((NGATHER, GATHER_ROWS), jnp.int32),
        pltpu.VMEM((NGATHER, GATHER_ROWS), jnp.int32),
        pltpu.VMEM((CHUNK, EMBED), jnp.float32),
        pltpu.VMEM((CHUNK, EMBED), jnp.float32),
        pltpu.VMEM((MAXLEN, EMBED), jnp.float32),
        pltpu.SemaphoreType.DMA,
        pltpu.SemaphoreType.DMA,
    ],
    compiler_params=pltpu.CompilerParams(use_tc_tiling_on_sc=False),
)
def _embed_kernel(idx_hbm, table_hbm, pos_hbm, out_hbm,
                  idx0, idx1, rows0, rows1, pos_v, gsem, osem):
    wid = lax.axis_index("s") * NC + lax.axis_index("c")
    base = wid * PER_W

    idx_bufs = (idx0, idx1)
    row_bufs = (rows0, rows1)

    pltpu.sync_copy(pos_hbm, pos_v)

    def fire_gathers(ci, idx_v, rows_v):
        off = base + ci * CHUNK
        pltpu.sync_copy(idx_hbm.at[pl.ds(off // GATHER_ROWS, NGATHER)], idx_v)
        for g in range(NGATHER):
            pltpu.async_copy(
                table_hbm.at[idx_v.at[g]],
                rows_v.at[pl.ds(g * GATHER_ROWS, GATHER_ROWS)],
                gsem,
            )

    def drain_gathers(idx_v, rows_v):
        for g in range(NGATHER):
            pltpu.make_async_copy(
                table_hbm.at[idx_v.at[g]],
                rows_v.at[pl.ds(g * GATHER_ROWS, GATHER_ROWS)],
                gsem,
            ).wait()

    def out_copy(ci, rows_v):
        off = base + ci * CHUNK
        return pltpu.make_async_copy(rows_v, out_hbm.at[pl.ds(off, CHUNK)], osem)

    def add_pos(ci, rows_v):
        off = base + ci * CHUNK

        def add_row(i, p):
            rows_v[i, pl.ds(0, 16)] = rows_v[i, pl.ds(0, 16)] + pos_v[p, pl.ds(0, 16)]
            rows_v[i, pl.ds(16, 16)] = rows_v[i, pl.ds(16, 16)] + pos_v[p, pl.ds(16, 16)]
            return lax.select(p == SEQ - 1, 0, p + 1)

        lax.fori_loop(0, CHUNK, add_row, lax.rem(off, SEQ), unroll=8)

    fire_gathers(0, idx0, rows0)

    def outer(k, carry):
        for j in range(2):
            ci = 2 * k + j
            idx_v, rows_v = idx_bufs[j], row_bufs[j]
            idx_n, rows_n = idx_bufs[1 - j], row_bufs[1 - j]

            @pl.when(ci > 0)
            def _():
                out_copy(ci - 1, rows_n).wait()

            @pl.when(ci + 1 < NCHUNK)
            def _():
                fire_gathers(ci + 1, idx_n, rows_n)

            drain_gathers(idx_v, rows_v)
            add_pos(ci, rows_v)
            out_copy(ci, rows_v).start()
        return carry

    lax.fori_loop(0, NCHUNK // 2, outer, None)

    out_copy(NCHUNK - 1, row_bufs[1]).wait()


def kernel(x, token_table, pos_table):
    idx = x.reshape(TOTAL // GATHER_ROWS, GATHER_ROWS).astype(jnp.int32)
    out = _embed_kernel(idx, token_table, pos_table)
    return out.reshape(BATCH, SEQ, EMBED)


# speedup vs baseline: 1.3828x; 1.1210x over previous
import functools

import jax
import jax.numpy as jnp
from jax import lax
from jax.experimental import pallas as pl
from jax.experimental.pallas import tpu as pltpu
from jax.experimental.pallas import tpu_sc as plsc

VOCAB = 1000000
MAXLEN = 200
EMBED = 32
BATCH = 4096
SEQ = 200

NC = 2
NS = 16
NW = NC * NS
BTILE = BATCH // NW
ETILES = EMBED // 8

_mesh = plsc.VectorSubcoreMesh(core_axis_name="c", subcore_axis_name="s")


@functools.partial(
    pl.kernel,
    out_type=jax.ShapeDtypeStruct((SEQ, ETILES, NW, 8 * BTILE), jnp.float32),
    mesh=_mesh,
    scratch_types=[
        pltpu.VMEM((SEQ, BTILE), jnp.int32),
        pltpu.VMEM((BTILE, EMBED), jnp.float32),
        pltpu.VMEM((BTILE, EMBED), jnp.float32),
        pltpu.VMEM((ETILES * 8 * BTILE,), jnp.float32),
        pltpu.VMEM((ETILES * 8 * BTILE,), jnp.float32),
        pltpu.VMEM((MAXLEN, EMBED), jnp.float32),
        pltpu.SemaphoreType.DMA,
        pltpu.SemaphoreType.DMA,
    ],
    compiler_params=pltpu.CompilerParams(
        use_tc_tiling_on_sc=False, needs_layout_passes=False
    ),
)
def _embed_kernel(xv_hbm, table_hbm, pos_hbm, out_hbm,
                  idx_all, rows0, rows1, tile0, tile1, pos_v, gsem, osem):
    wid = lax.axis_index("s") * NC + lax.axis_index("c")
    rows_bufs = (rows0, rows1)
    tile_bufs = (tile0, tile1)

    pltpu.sync_copy(pos_hbm, pos_v)
    pltpu.sync_copy(xv_hbm.at[:, wid], idx_all)

    lane_e = lax.iota(jnp.int32, 16) * BTILE
    lane_e_hi = lane_e + 16 * BTILE

    def gather(s, rows_v):
        return pltpu.make_async_copy(table_hbm.at[idx_all.at[s]], rows_v, gsem)

    def out_write(s, tile_v, et):
        return pltpu.make_async_copy(
            tile_v.at[pl.ds(et * 8 * BTILE, 8 * BTILE)],
            out_hbm.at[s, et, wid],
            osem,
        )

    def transpose_add(s, rows_v, tile_v):
        pos_lo = pos_v[s, pl.ds(0, 16)]
        pos_hi = pos_v[s, pl.ds(16, 16)]

        def body(b, carry):
            plsc.store_scatter(tile_v, [lane_e + b],
                               rows_v[b, pl.ds(0, 16)] + pos_lo)
            plsc.store_scatter(tile_v, [lane_e_hi + b],
                               rows_v[b, pl.ds(16, 16)] + pos_hi)
            return carry

        lax.fori_loop(0, BTILE, body, None, unroll=8)

    gather(0, rows0).start()

    def outer(k, carry):
        for j in range(2):
            s = 2 * k + j
            rows_v, tile_v = rows_bufs[j], tile_bufs[j]

            @pl.when(k >= 1)
            def _():
                for et in range(ETILES):
                    out_write(s - 2, tile_v, et).wait()

            if j == 0:
                gather(s + 1, rows_bufs[1]).start()
            else:
                @pl.when(k < SEQ // 2 - 1)
                def _():
                    gather(s + 1, rows_bufs[0]).start()

            gather(s, rows_v).wait()
            transpose_add(s, rows_v, tile_v)
            for et in range(ETILES):
                out_write(s, tile_v, et).start()
        return carry

    lax.fori_loop(0, SEQ // 2, outer, None)

    for et in range(ETILES):
        out_write(SEQ - 2, tile0, et).wait()
    for et in range(ETILES):
        out_write(SEQ - 1, tile1, et).wait()


def kernel(x, token_table, pos_table):
    xv = x.T.reshape(SEQ, NW, BTILE).astype(jnp.int32)
    out = _embed_kernel(xv, token_table, pos_table)
    out = out.reshape(SEQ, ETILES, NW, 8, BTILE)
    return out.transpose(2, 4, 0, 1, 3).reshape(BATCH, SEQ, EMBED)
